# baseline (speedup 1.0000x reference)
import jax
import jax.numpy as jnp
from jax.experimental import pallas as pl
from jax.experimental.pallas import tpu as pltpu

_INPUT_DIM = 8192
_OUT_DIM = 6
_L = 8
_D_MODEL = 32
_N = 16
_K_CONV = 4
_DIN = 64
_BATCH = 64
_BL = _BATCH * _L
_CORES = 2
_BH = _BATCH // _CORES
_RH = _BH * _L
_KBLK = 1024
_KB = _INPUT_DIM // _KBLK


def _mamba_layer(xz, conv_w, conv_b, x_proj_w, dt_proj_w, dt_proj_b,
                 A_t, d_skip, last):
    f32 = jnp.float32
    x3 = xz[:, :_DIN].reshape(_BH, _L, _DIN)
    z3 = xz[:, _DIN:].reshape(_BH, _L, _DIN)

    acc = (conv_b.reshape(1, 1, _DIN)
           + conv_w[_K_CONV - 1:_K_CONV, :].reshape(1, 1, _DIN) * x3)
    for k in range(_K_CONV - 1):
        s = _K_CONV - 1 - k
        shifted = jnp.concatenate(
            [jnp.zeros((_BH, s, _DIN), f32), x3[:, :_L - s, :]], axis=1)
        acc = acc + conv_w[k:k + 1, :].reshape(1, 1, _DIN) * shifted
    xc3 = acc * jax.nn.sigmoid(acc)
    xc2 = xc3.reshape(_RH, _DIN)

    dbc = jnp.dot(xc2, x_proj_w, preferred_element_type=f32)
    dt_lin = jnp.dot(dbc[:, :2], dt_proj_w, preferred_element_type=f32)
    delta3 = jax.nn.softplus(dt_lin + dt_proj_b).reshape(_BH, _L, _DIN)
    Bm = dbc[:, 2:2 + _N].reshape(_BH, _L, _N)
    Cm = dbc[:, 2 + _N:2 + 2 * _N].reshape(_BH, _L, _N)

    dA = jnp.exp(delta3[:, :, None, :] * A_t[None, None, :, :])
    dBu = Bm[:, :, :, None] * (delta3 * xc3)[:, :, None, :]

    h = jnp.zeros((_BH, _N, _DIN), f32)
    if last:
        for t in range(_L):
            h = dA[:, t] * h + dBu[:, t]
        y = jnp.sum(h * Cm[:, _L - 1, :, None], axis=1)
        xc_l = xc3[:, _L - 1]
        z_l = z3[:, _L - 1]
        return (y + d_skip * xc_l) * (z_l * jax.nn.sigmoid(z_l))

    ys = []
    for t in range(_L):
        h = dA[:, t] * h + dBu[:, t]
        ys.append(jnp.sum(h * Cm[:, t, :, None], axis=1))
    y3 = jnp.stack(ys, axis=1)
    y3 = (y3 + d_skip.reshape(1, 1, _DIN) * xc3) * (z3 * jax.nn.sigmoid(z3))
    return y3.reshape(_RH, _DIN)


def _fused_kernel(x_ref, emb_w_ref, emb_b_ref, head_w_ref, head_b_ref,
                  ip0, cw0, cb0, xp0, dw0, db0, op0, a0, d0,
                  ip1, cw1, cb1, xp1, dw1, db1, op1, a1, d1,
                  o_ref, acc_ref):
    f32 = jnp.float32
    k = pl.program_id(1)

    @pl.when(k == 0)
    def _init():
        acc_ref[...] = jnp.zeros_like(acc_ref)

    acc_ref[...] += jnp.dot(x_ref[...].reshape(_RH, _KBLK), emb_w_ref[...],
                            preferred_element_type=f32)

    @pl.when(k == _KB - 1)
    def _tail():
        e = acc_ref[...] + emb_b_ref[...]
        xz = jnp.dot(e, ip0[...], preferred_element_type=f32)
        y2 = _mamba_layer(xz, cw0[...], cb0[...], xp0[...], dw0[...],
                          db0[...], a0[...], d0[...], last=False)
        xz1 = jnp.dot(jnp.dot(y2, op0[...], preferred_element_type=f32),
                      ip1[...], preferred_element_type=f32)
        y_last = _mamba_layer(xz1, cw1[...], cb1[...], xp1[...], dw1[...],
                              db1[...], a1[...], d1[...], last=True)
        o = jnp.dot(jnp.dot(y_last, op1[...], preferred_element_type=f32),
                    head_w_ref[...], preferred_element_type=f32)
        o_ref[...] = o + head_b_ref[...]


def _small(shape):
    return pl.BlockSpec(shape, lambda i, k: (0,) * len(shape))


def kernel(x, emb_w, emb_b, head_w, head_b,
           l0_in_proj_w, l0_conv_w, l0_conv_b, l0_x_proj_w, l0_dt_proj_w,
           l0_dt_proj_b, l0_out_proj_w, l0_A_t, l0_D,
           l1_in_proj_w, l1_conv_w, l1_conv_b, l1_x_proj_w, l1_dt_proj_w,
           l1_dt_proj_b, l1_out_proj_w, l1_A_t, l1_D):
    operands = (x, emb_w, emb_b, head_w, head_b,
                l0_in_proj_w, l0_conv_w, l0_conv_b, l0_x_proj_w, l0_dt_proj_w,
                l0_dt_proj_b, l0_out_proj_w, l0_A_t, l0_D,
                l1_in_proj_w, l1_conv_w, l1_conv_b, l1_x_proj_w, l1_dt_proj_w,
                l1_dt_proj_b, l1_out_proj_w, l1_A_t, l1_D)
    in_specs = [
        pl.BlockSpec((_BH, _L, _KBLK), lambda i, k: (i, 0, k)),
        pl.BlockSpec((_KBLK, _D_MODEL), lambda i, k: (k, 0)),
    ] + [_small(op.shape) for op in operands[2:]]

    return pl.pallas_call(
        _fused_kernel,
        out_shape=jax.ShapeDtypeStruct((_BATCH, _OUT_DIM), jnp.float32),
        grid=(_CORES, _KB),
        in_specs=in_specs,
        out_specs=pl.BlockSpec((_BH, _OUT_DIM), lambda i, k: (i, 0)),
        scratch_shapes=[pltpu.VMEM((_RH, _D_MODEL), jnp.float32)],
        compiler_params=pltpu.CompilerParams(
            dimension_semantics=("parallel", "arbitrary")),
    )(*operands)


# speedup vs baseline: 2.1681x; 2.1681x over previous
import jax
import jax.numpy as jnp
from jax.experimental import pallas as pl
from jax.experimental.pallas import tpu as pltpu

_INPUT_DIM = 8192
_OUT_DIM = 6
_L = 8
_D_MODEL = 32
_N = 16
_K_CONV = 4
_DIN = 64
_ND = _N * _DIN
_BATCH = 64
_CORES = 2
_BH = _BATCH // _CORES
_RH = _BH * _L
_XCH = 4
_SEQ_PER_CH = _BH // _XCH


def _iota(shape, dim):
    return jax.lax.broadcasted_iota(jnp.int32, shape, dim)


def _perm_tmajor():
    r = _iota((_RH, _RH), 0)
    c = _iota((_RH, _RH), 1)
    src = (r % _BH) * _L + r // _BH
    return jnp.where(c == src, 1.0, 0.0).astype(jnp.float32)


def _expand_bc():
    r = _iota((2 + 2 * _N, 2 * _ND), 0)
    c = _iota((2 + 2 * _N, 2 * _ND), 1)
    n = (c % _ND) // _DIN
    off = jnp.where(c < _ND, 2, 2 + _N)
    return jnp.where(r == off + n, 1.0, 0.0).astype(jnp.float32)


def _expand_d():
    r = _iota((_DIN, _ND), 0)
    c = _iota((_DIN, _ND), 1)
    return jnp.where(r == c % _DIN, 1.0, 0.0).astype(jnp.float32)


def _flatten_rows(a, rows):
    return jnp.concatenate([a[j:j + 1, :] for j in range(rows)], axis=1)


def _sum_over_n(v):
    s = v[:, :128]
    for j in range(1, _ND // 128):
        s = s + v[:, j * 128:(j + 1) * 128]
    return s[:, :_DIN] + s[:, _DIN:]


def _mamba_layer_tmajor(xz, conv_w, conv_b, x_proj_w, dt_proj_w, dt_proj_b,
                        a_row, d_skip, e_bc, t_d, last):
    f32 = jnp.float32
    xp = xz[:, :_DIN]
    z = xz[:, _DIN:]

    acc = conv_b + conv_w[_K_CONV - 1:_K_CONV, :] * xp
    for k in range(_K_CONV - 1):
        s = (_K_CONV - 1 - k) * _BH
        shifted = jnp.concatenate(
            [jnp.zeros((s, _DIN), f32), xp[:_RH - s, :]], axis=0)
        acc = acc + conv_w[k:k + 1, :] * shifted
    xc = acc * jax.nn.sigmoid(acc)

    dbc = jnp.dot(xc, x_proj_w, preferred_element_type=f32)
    dt_lin = jnp.dot(dbc[:, :2], dt_proj_w, preferred_element_type=f32)
    delta = jax.nn.softplus(dt_lin + dt_proj_b)

    bc4 = jnp.dot(dbc, e_bc, preferred_element_type=f32)
    b4 = bc4[:, :_ND]
    c4 = bc4[:, _ND:]
    dd = jnp.dot(jnp.concatenate([delta, delta * xc], axis=0), t_d,
                 preferred_element_type=f32)
    da = jnp.exp(dd[:_RH] * a_row)
    dbu = b4 * dd[_RH:]

    h = jnp.zeros((_BH, _ND), f32)
    if last:
        for t in range(_L):
            lo = t * _BH
            h = da[lo:lo + _BH] * h + dbu[lo:lo + _BH]
        lo = (_L - 1) * _BH
        y = _sum_over_n(h * c4[lo:lo + _BH])
        xc_l = xc[lo:lo + _BH]
        z_l = z[lo:lo + _BH]
        return (y + d_skip * xc_l) * (z_l * jax.nn.sigmoid(z_l))

    ys = []
    for t in range(_L):
        lo = t * _BH
        h = da[lo:lo + _BH] * h + dbu[lo:lo + _BH]
        ys.append(_sum_over_n(h * c4[lo:lo + _BH]))
    y = jnp.concatenate(ys, axis=0)
    y = (y + d_skip * xc) * (z * jax.nn.sigmoid(z))
    return y


def _fused_kernel(x_h, emb_w_h, emb_b_h, head_w_h, head_b_h,
                  ip0_h, cw0_h, cb0_h, xp0_h, dw0_h, db0_h, op0_h, a0_h, d0_h,
                  ip1_h, cw1_h, cb1_h, xp1_h, dw1_h, db1_h, op1_h, a1_h, d1_h,
                  o_ref,
                  xfull, ebuf, emb_b_v, head_w_v, head_b_v,
                  ip0_v, cw0_v, cb0_v, xp0_v, dw0_v, db0_v, op0_v, a0_v, d0_v,
                  ip1_v, cw1_v, cb1_v, xp1_v, dw1_v, db1_v, op1_v, a1_v, d1_v,
                  sems):
    f32 = jnp.float32
    i = pl.program_id(0)

    def xcp(j):
        return pltpu.make_async_copy(
            x_h.at[pl.ds(i * _BH + j * _SEQ_PER_CH, _SEQ_PER_CH)],
            xfull.at[pl.ds(j * _SEQ_PER_CH, _SEQ_PER_CH)],
            sems.at[j])

    ecp = pltpu.make_async_copy(emb_w_h, ebuf, sems.at[_XCH])
    wpairs = [(emb_b_h, emb_b_v), (head_w_h, head_w_v), (head_b_h, head_b_v),
              (ip0_h, ip0_v), (cw0_h, cw0_v), (cb0_h, cb0_v), (xp0_h, xp0_v),
              (dw0_h, dw0_v), (db0_h, db0_v), (op0_h, op0_v), (a0_h, a0_v),
              (d0_h, d0_v),
              (ip1_h, ip1_v), (cw1_h, cw1_v), (cb1_h, cb1_v), (xp1_h, xp1_v),
              (dw1_h, dw1_v), (db1_h, db1_v), (op1_h, op1_v), (a1_h, a1_v),
              (d1_h, d1_v)]
    wcps = [pltpu.make_async_copy(src, dst, sems.at[_XCH + 1 + j])
            for j, (src, dst) in enumerate(wpairs)]

    ecp.start()
    for j in range(_XCH):
        xcp(j).start()
    for cp in wcps:
        cp.start()

    ecp.wait()
    emb_w = ebuf[...]
    chunks = []
    for j in range(_XCH):
        xcp(j).wait()
        xc_rows = xfull[j * _SEQ_PER_CH:(j + 1) * _SEQ_PER_CH]
        xm = xc_rows.reshape(_SEQ_PER_CH * _L, _INPUT_DIM)
        chunks.append(jnp.dot(xm, emb_w, preferred_element_type=f32))
    for cp in wcps:
        cp.wait()

    e = jnp.concatenate(chunks, axis=0) + emb_b_v[...]

    e_t = jnp.dot(_perm_tmajor(), e, preferred_element_type=f32)
    e_bc = _expand_bc()
    t_d = _expand_d()
    a0_row = _flatten_rows(a0_v[...], _N)
    a1_row = _flatten_rows(a1_v[...], _N)

    xz = jnp.dot(e_t, ip0_v[...], preferred_element_type=f32)
    y = _mamba_layer_tmajor(xz, cw0_v[...], cb0_v[...], xp0_v[...], dw0_v[...],
                            db0_v[...], a0_row, d0_v[...],
                            e_bc, t_d, last=False)
    xz1 = jnp.dot(jnp.dot(y, op0_v[...], preferred_element_type=f32),
                  ip1_v[...], preferred_element_type=f32)
    y_last = _mamba_layer_tmajor(xz1, cw1_v[...], cb1_v[...], xp1_v[...],
                                 dw1_v[...], db1_v[...], a1_row, d1_v[...],
                                 e_bc, t_d, last=True)
    o = jnp.dot(jnp.dot(y_last, op1_v[...], preferred_element_type=f32),
                head_w_v[...], preferred_element_type=f32)
    o_ref[...] = o + head_b_v[...]


def kernel(x, emb_w, emb_b, head_w, head_b,
           l0_in_proj_w, l0_conv_w, l0_conv_b, l0_x_proj_w, l0_dt_proj_w,
           l0_dt_proj_b, l0_out_proj_w, l0_A_t, l0_D,
           l1_in_proj_w, l1_conv_w, l1_conv_b, l1_x_proj_w, l1_dt_proj_w,
           l1_dt_proj_b, l1_out_proj_w, l1_A_t, l1_D):
    operands = (x, emb_w, emb_b, head_w, head_b,
                l0_in_proj_w, l0_conv_w, l0_conv_b, l0_x_proj_w, l0_dt_proj_w,
                l0_dt_proj_b, l0_out_proj_w, l0_A_t, l0_D,
                l1_in_proj_w, l1_conv_w, l1_conv_b, l1_x_proj_w, l1_dt_proj_w,
                l1_dt_proj_b, l1_out_proj_w, l1_A_t, l1_D)
    small_shapes = [op.shape for op in operands[2:]]

    return pl.pallas_call(
        _fused_kernel,
        out_shape=jax.ShapeDtypeStruct((_BATCH, _OUT_DIM), jnp.float32),
        grid=(_CORES,),
        in_specs=[pl.BlockSpec(memory_space=pl.ANY)] * len(operands),
        out_specs=pl.BlockSpec((_BH, _OUT_DIM), lambda i: (i, 0)),
        scratch_shapes=(
            [pltpu.VMEM((_BH, _L, _INPUT_DIM), jnp.float32),
             pltpu.VMEM((_INPUT_DIM, _D_MODEL), jnp.float32)]
            + [pltpu.VMEM(s, jnp.float32) for s in small_shapes]
            + [pltpu.SemaphoreType.DMA((_XCH + 1 + len(small_shapes),))]
        ),
        compiler_params=pltpu.CompilerParams(
            dimension_semantics=("parallel",)),
    )(*operands)
